# Initial kernel scaffold; baseline (speedup 1.0000x reference)
import jax
import jax.numpy as jnp
from jax.experimental import pallas as pl


def kernel(x, edge_index, batch, W1_root, W1_nbr, b1, W2_root, W2_nbr, b2, Wlin, blin):
    raise NotImplementedError("write your pallas kernel here")


# speedup vs baseline: 1.0267x; 1.0267x over previous
import jax
import jax.numpy as jnp
from jax.experimental import pallas as pl

N = 10000
D = 128
E = 320000
R = 1000


def _tc1_body(a_ref, dg_ref, x_ref, wr_ref, wn_ref, b_ref, o_ref):
    deg = jnp.maximum(dg_ref[...], 1.0)
    nbr = a_ref[...] / deg
    o_ref[...] = (
        jnp.dot(x_ref[...], wr_ref[...], preferred_element_type=jnp.float32)
        + jnp.dot(nbr, wn_ref[...], preferred_element_type=jnp.float32)
        + b_ref[...])


def _tc2_body(a_ref, dg_ref, x1_ref, wr_ref, wn_ref, b_ref,
              wa_ref, wb_ref, bl_ref, o_ref):
    deg = jnp.maximum(dg_ref[...], 1.0)
    nbr = a_ref[...] / deg
    mu = jnp.mean(nbr, axis=-1, keepdims=True)
    var = jnp.mean((nbr - mu) ** 2, axis=-1, keepdims=True)
    nbrn = (nbr - mu) / jnp.sqrt(var + 1e-5)
    x1 = x1_ref[...]
    x2 = (jnp.dot(x1, wr_ref[...], preferred_element_type=jnp.float32)
          + jnp.dot(nbrn, wn_ref[...], preferred_element_type=jnp.float32)
          + b_ref[...])
    o_ref[...] = jnp.maximum(
        jnp.dot(x1, wa_ref[...], preferred_element_type=jnp.float32)
        + jnp.dot(x2, wb_ref[...], preferred_element_type=jnp.float32)
        + bl_ref[...], 0.0)


_A_SPEC = pl.BlockSpec((R, D), lambda i: (i, 0))
_DG_SPEC = pl.BlockSpec((R, 1), lambda i: (i, 0))
_W_SPEC = pl.BlockSpec((D, D), lambda i: (0, 0))
_B_SPEC = pl.BlockSpec((1, D), lambda i: (0, 0))


def _tc1(agg, deg, x, wr, wn, b):
    return pl.pallas_call(
        _tc1_body,
        grid=(N // R,),
        in_specs=[_A_SPEC, _DG_SPEC, _A_SPEC, _W_SPEC, _W_SPEC, _B_SPEC],
        out_specs=_A_SPEC,
        out_shape=jax.ShapeDtypeStruct((N, D), jnp.float32),
    )(agg, deg, x, wr, wn, b)


def _tc2(agg, deg, x1, wr, wn, b, wa, wb, bl):
    return pl.pallas_call(
        _tc2_body,
        grid=(N // R,),
        in_specs=[_A_SPEC, _DG_SPEC, _A_SPEC, _W_SPEC, _W_SPEC, _B_SPEC,
                  _W_SPEC, _W_SPEC, _B_SPEC],
        out_specs=_A_SPEC,
        out_shape=jax.ShapeDtypeStruct((N, D), jnp.float32),
    )(agg, deg, x1, wr, wn, b, wa, wb, bl)


def kernel(x, edge_index, batch, W1_root, W1_nbr, b1, W2_root, W2_nbr, b2,
           Wlin, blin):
    src = edge_index[0]
    dst = edge_index[1]
    deg = jax.ops.segment_sum(jnp.ones((E,), jnp.float32), dst,
                              num_segments=N)[:, None]
    agg1 = jax.ops.segment_sum(jnp.take(x, src, axis=0), dst, num_segments=N)
    x1 = _tc1(agg1, deg, x, W1_root, W1_nbr, b1.reshape(1, D))
    agg2 = jax.ops.segment_sum(jnp.take(x1, src, axis=0), dst, num_segments=N)
    out = _tc2(agg2, deg, x1, W2_root, W2_nbr, b2.reshape(1, D),
               Wlin[:D], Wlin[D:], blin.reshape(1, D))
    return out


# speedup vs baseline: 1.0918x; 1.0633x over previous
import jax
import jax.numpy as jnp
from jax.experimental import pallas as pl

N = 10000
D = 128
E = 320000
R = 1000


def _tc1_body(a_ref, dg_ref, x_ref, wr_ref, wn_ref, b_ref, o_ref):
    deg = jnp.maximum(dg_ref[...], 1.0)
    nbr = a_ref[...] / deg
    o_ref[...] = (
        jnp.dot(x_ref[...], wr_ref[...], preferred_element_type=jnp.float32)
        + jnp.dot(nbr, wn_ref[...], preferred_element_type=jnp.float32)
        + b_ref[...])


def _tc2_body(a_ref, dg_ref, x1_ref, wr_ref, wn_ref, b_ref,
              wa_ref, wb_ref, bl_ref, o_ref):
    deg = jnp.maximum(dg_ref[...], 1.0)
    nbr = a_ref[...] / deg
    mu = jnp.mean(nbr, axis=-1, keepdims=True)
    var = jnp.mean((nbr - mu) ** 2, axis=-1, keepdims=True)
    nbrn = (nbr - mu) / jnp.sqrt(var + 1e-5)
    x1 = x1_ref[...]
    x2 = (jnp.dot(x1, wr_ref[...], preferred_element_type=jnp.float32)
          + jnp.dot(nbrn, wn_ref[...], preferred_element_type=jnp.float32)
          + b_ref[...])
    o_ref[...] = jnp.maximum(
        jnp.dot(x1, wa_ref[...], preferred_element_type=jnp.float32)
        + jnp.dot(x2, wb_ref[...], preferred_element_type=jnp.float32)
        + bl_ref[...], 0.0)


_A_SPEC = pl.BlockSpec((R, D), lambda i: (i, 0))
_DG_SPEC = pl.BlockSpec((R, 1), lambda i: (i, 0))
_W_SPEC = pl.BlockSpec((D, D), lambda i: (0, 0))
_B_SPEC = pl.BlockSpec((1, D), lambda i: (0, 0))


def _tc1(agg, deg, x, wr, wn, b):
    return pl.pallas_call(
        _tc1_body,
        grid=(N // R,),
        in_specs=[_A_SPEC, _DG_SPEC, _A_SPEC, _W_SPEC, _W_SPEC, _B_SPEC],
        out_specs=_A_SPEC,
        out_shape=jax.ShapeDtypeStruct((N, D), jnp.float32),
    )(agg, deg, x, wr, wn, b)


def _tc2(agg, deg, x1, wr, wn, b, wa, wb, bl):
    return pl.pallas_call(
        _tc2_body,
        grid=(N // R,),
        in_specs=[_A_SPEC, _DG_SPEC, _A_SPEC, _W_SPEC, _W_SPEC, _B_SPEC,
                  _W_SPEC, _W_SPEC, _B_SPEC],
        out_specs=_A_SPEC,
        out_shape=jax.ShapeDtypeStruct((N, D), jnp.float32),
    )(agg, deg, x1, wr, wn, b, wa, wb, bl)


def kernel(x, edge_index, batch, W1_root, W1_nbr, b1, W2_root, W2_nbr, b2,
           Wlin, blin):
    order = jnp.argsort(edge_index[1])
    src = jnp.take(edge_index[0], order)
    dst = jnp.take(edge_index[1], order)
    deg = jax.ops.segment_sum(jnp.ones((E,), jnp.float32), dst,
                              num_segments=N, indices_are_sorted=True)[:, None]
    agg1 = jax.ops.segment_sum(jnp.take(x, src, axis=0), dst, num_segments=N,
                               indices_are_sorted=True)
    x1 = _tc1(agg1, deg, x, W1_root, W1_nbr, b1.reshape(1, D))
    agg2 = jax.ops.segment_sum(jnp.take(x1, src, axis=0), dst, num_segments=N,
                               indices_are_sorted=True)
    out = _tc2(agg2, deg, x1, W2_root, W2_nbr, b2.reshape(1, D),
               Wlin[:D], Wlin[D:], blin.reshape(1, D))
    return out
